# baseline (speedup 1.0000x reference)
import functools

import jax
import jax.numpy as jnp
from jax import lax
from jax.experimental import pallas as pl
from jax.experimental.pallas import tpu as pltpu
from jax.experimental.pallas import tpu_sc as plsc

NN = 50000
EE = 1600000
GG = 64
CC = 40

NTILES = 32
EPT = EE // NTILES
GCB = 1000
NCHUNK = 10
ECH = EE // NCHUNK
SCB = 1600

_SC_MESH = plsc.VectorSubcoreMesh(
    core_axis_name="c", subcore_axis_name="s", num_cores=2, num_subcores=16)
_SC_PARAMS = pltpu.CompilerParams(use_tc_tiling_on_sc=False,
                                  needs_layout_passes=False)

F32 = jnp.float32


def _gather_body(tsrc, tdst, src_i, dst_i, out_s, out_d,
                 sidx, didx, srows, drows, tdst_sh, sem1, sem2):
    wid = lax.axis_index("s") * 2 + lax.axis_index("c")
    base = wid * EPT

    @pl.when(lax.axis_index("s") == 0)
    def _():
        pltpu.sync_copy(tdst, tdst_sh)

    plsc.subcore_barrier()

    nch = EPT // GCB

    def _cp(k, b):
        off = base + k * GCB
        pltpu.sync_copy(src_i.at[pl.ds(off, GCB)], sidx.at[b])
        pltpu.sync_copy(dst_i.at[pl.ds(off, GCB)], didx.at[b])
        return (pltpu.make_async_copy(tsrc.at[sidx.at[b]], srows.at[b], sem1),
                pltpu.make_async_copy(tdst_sh.at[didx.at[b]], drows.at[b],
                                      sem2))

    def _start(k, b):
        c1, c2 = _cp(k, b)
        c1.start()
        c2.start()

    _start(0, 0)

    def step(k2, carry):
        for b in (0, 1):
            k = 2 * k2 + b
            c1 = pltpu.make_async_copy(tsrc.at[sidx.at[b]], srows.at[b], sem1)
            c2 = pltpu.make_async_copy(tdst_sh.at[didx.at[b]], drows.at[b],
                                       sem2)
            c1.wait()
            c2.wait()

            @pl.when(k + 1 < nch)
            def _():
                _start(k + 1, 1 - b)

            off = base + k * GCB
            pltpu.sync_copy(srows.at[b], out_s.at[pl.ds(off, GCB)])
            pltpu.sync_copy(drows.at[b], out_d.at[pl.ds(off, GCB)])
        return carry

    lax.fori_loop(0, nch // 2, step, 0)


_gather_call = pl.kernel(
    _gather_body,
    out_type=(
        jax.ShapeDtypeStruct((EE, 8), F32),
        jax.ShapeDtypeStruct((EE, 8), F32),
    ),
    mesh=_SC_MESH,
    compiler_params=_SC_PARAMS,
    scratch_types=[
        pltpu.VMEM((2, GCB), jnp.int32),
        pltpu.VMEM((2, GCB), jnp.int32),
        pltpu.VMEM((2, GCB, 8), F32),
        pltpu.VMEM((2, GCB, 8), F32),
        pltpu.VMEM_SHARED((NN, 8), F32),
        pltpu.SemaphoreType.DMA,
        pltpu.SemaphoreType.DMA,
    ],
)


def _scatter_body(dst_i, msg2, out, didx, mvals, agg):
    wid = lax.axis_index("s") * 2 + lax.axis_index("c")

    neg = jnp.full((16,), -jnp.inf, F32)

    def init_step(i, carry):
        agg[pl.ds(i * 16, 16)] = neg
        return carry

    lax.fori_loop(0, NN // 16, init_step, 0)

    @pl.when(wid < 3 * NCHUNK)
    def _():
        ch = wid // NCHUNK
        chunk = wid % NCHUNK

        def outer(k, carry):
            off = chunk * ECH + k * SCB
            pltpu.sync_copy(dst_i.at[pl.ds(off, SCB)], didx)
            pltpu.sync_copy(msg2.at[pl.ds(off // 16, SCB // 16), :], mvals)

            def inner(g, c2):
                d16 = didx[pl.ds(g * 16, 16)]
                m16 = mvals[g, pl.ds(ch * 16, 16)]
                cnt, _ = plsc.scan_count(d16)
                c0 = cnt[0]
                cur = plsc.load_gather(agg, [d16])
                newv = jnp.maximum(cur, m16)
                plsc.store_scatter(agg, [d16], newv, mask=cnt == c0)

                @pl.when(jnp.any(cnt > c0))
                def _():
                    rmax = lax.reduce_max(cnt, (0,))

                    def rbody(r, c3):
                        cur2 = plsc.load_gather(agg, [d16])
                        plsc.store_scatter(agg, [d16],
                                           jnp.maximum(cur2, m16),
                                           mask=cnt == r)
                        return c3

                    lax.fori_loop(c0 + 1, rmax + 1, rbody, 0)

                return c2

            lax.fori_loop(0, SCB // 16, inner, 0)
            return carry

        lax.fori_loop(0, ECH // SCB, outer, 0)

    pltpu.sync_copy(agg, out.at[wid])


_scatter_call = pl.kernel(
    _scatter_body,
    out_type=jax.ShapeDtypeStruct((NTILES, NN), F32),
    mesh=_SC_MESH,
    compiler_params=_SC_PARAMS,
    scratch_types=[
        pltpu.VMEM((SCB,), jnp.int32),
        pltpu.VMEM((SCB // 16, 128), F32),
        pltpu.VMEM((NN,), F32),
    ],
)


def _dot_nt(a, b):
    return lax.dot_general(a, b, (((1,), (1,)), ((), ())),
                           preferred_element_type=F32)


def _node_h_body(x_ref, w1, b1, w2, b2, o_ref):
    xb = x_ref[...]
    hid = jnp.maximum(_dot_nt(xb, w1[...]) + b1[...], 0.0)
    o_ref[...] = _dot_nt(hid, w2[...]) + b2[...]


BN_A = 8192


def _node_h(x, w1, b1, w2, b2):
    grid = (NN + BN_A - 1) // BN_A
    return pl.pallas_call(
        _node_h_body,
        grid=(grid,),
        in_specs=[
            pl.BlockSpec((BN_A, 3), lambda i: (i, 0)),
            pl.BlockSpec((64, 3), lambda i: (0, 0)),
            pl.BlockSpec((1, 64), lambda i: (0, 0)),
            pl.BlockSpec((3, 64), lambda i: (0, 0)),
            pl.BlockSpec((1, 3), lambda i: (0, 0)),
        ],
        out_specs=pl.BlockSpec((BN_A, 3), lambda i: (i, 0)),
        out_shape=jax.ShapeDtypeStruct((NN, 3), F32),
    )(x, w1, b1, w2, b2)


BR = 256
ER = EE // 16


def _edge_mlp_body(s_ref, d_ref, w1, b1, w2, b2, o_ref):
    feat = s_ref[...] - d_ref[...]
    hid = lax.dot_general(feat, w1[...], (((1,), (0,)), ((), ())),
                          preferred_element_type=F32)
    hid = jnp.maximum(hid + b1[...], 0.0)
    msg = lax.dot_general(hid, w2[...], (((1,), (0,)), ((), ())),
                          preferred_element_type=F32)
    o_ref[...] = msg + b2[...]


def _edge_mlp(srows2, drows2, w1bd, b1t, w2pl, b2pl):
    grid = (ER + BR - 1) // BR
    return pl.pallas_call(
        _edge_mlp_body,
        grid=(grid,),
        in_specs=[
            pl.BlockSpec((BR, 128), lambda i: (i, 0)),
            pl.BlockSpec((BR, 128), lambda i: (i, 0)),
            pl.BlockSpec((128, 1024), lambda i: (0, 0)),
            pl.BlockSpec((1, 1024), lambda i: (0, 0)),
            pl.BlockSpec((1024, 128), lambda i: (0, 0)),
            pl.BlockSpec((1, 128), lambda i: (0, 0)),
        ],
        out_specs=pl.BlockSpec((BR, 128), lambda i: (i, 0)),
        out_shape=jax.ShapeDtypeStruct((ER, 128), F32),
    )(srows2, drows2, w1bd, b1t, w2pl, b2pl)


def _merge_mlp_body(p_ref, wg1, bg1, wg2, bg2, wh1, bh1, wh2, bh2, o_ref,
                    *, do_relu, do_h):
    p = p_ref[...]
    a0 = jnp.max(p[0:10], axis=0)
    a1 = jnp.max(p[10:20], axis=0)
    a2 = jnp.max(p[20:30], axis=0)
    agg_t = jnp.concatenate(
        [a0.reshape(1, -1), a1.reshape(1, -1), a2.reshape(1, -1)], axis=0)
    agg_t = jnp.where(jnp.isfinite(agg_t), agg_t, 0.0)
    eye3 = jnp.eye(3, dtype=F32)
    agg = lax.dot_general(agg_t, eye3, (((0,), (0,)), ((), ())),
                          preferred_element_type=F32)
    hid = jnp.maximum(_dot_nt(agg, wg1[...]) + bg1[...], 0.0)
    g = _dot_nt(hid, wg2[...]) + bg2[...]
    if do_relu:
        g = jnp.maximum(g, 0.0)
    if do_h:
        hid2 = jnp.maximum(_dot_nt(g, wh1[...]) + bh1[...], 0.0)
        g = _dot_nt(hid2, wh2[...]) + bh2[...]
    o_ref[...] = g


BN_B = 2048


def _merge_mlp(partial, wg1, bg1, wg2, bg2, wh1, bh1, wh2, bh2,
               do_relu, do_h):
    grid = (NN + BN_B - 1) // BN_B
    body = functools.partial(_merge_mlp_body, do_relu=do_relu, do_h=do_h)
    return pl.pallas_call(
        body,
        grid=(grid,),
        in_specs=[
            pl.BlockSpec((NTILES, BN_B), lambda i: (0, i)),
            pl.BlockSpec((64, 3), lambda i: (0, 0)),
            pl.BlockSpec((1, 64), lambda i: (0, 0)),
            pl.BlockSpec((3, 64), lambda i: (0, 0)),
            pl.BlockSpec((1, 3), lambda i: (0, 0)),
            pl.BlockSpec((64, 3), lambda i: (0, 0)),
            pl.BlockSpec((1, 64), lambda i: (0, 0)),
            pl.BlockSpec((3, 64), lambda i: (0, 0)),
            pl.BlockSpec((1, 3), lambda i: (0, 0)),
        ],
        out_specs=pl.BlockSpec((BN_B, 3), lambda i: (i, 0)),
        out_shape=jax.ShapeDtypeStruct((NN, 3), F32),
    )(partial, wg1, bg1, wg2, bg2, wh1, bh1, wh2, bh2)


BN_P = 2000
NBLK_P = NN // BN_P


def _pool_body(h_ref, b_ref, wl, bl, o_ref, acc):
    i = pl.program_id(0)

    @pl.when(i == 0)
    def _():
        acc[...] = jnp.zeros_like(acc)

    h = h_ref[...]
    bt = b_ref[0, 0, :]
    iota = lax.broadcasted_iota(jnp.int32, (GG, BN_P), 0)
    oneh = (iota == bt[None, :]).astype(F32)
    sums = lax.dot_general(oneh, h, (((1,), (0,)), ((), ())),
                           preferred_element_type=F32)
    cnts = jnp.sum(oneh, axis=1).reshape(GG, 1)
    acc[:, 0:3] += sums
    acc[:, 3:4] += cnts

    @pl.when(i == NBLK_P - 1)
    def _():
        pooled = acc[:, 0:3] / jnp.maximum(acc[:, 3:4], 1.0)
        logits = _dot_nt(pooled, wl[...]) + bl[...]
        m = jnp.max(logits, axis=1, keepdims=True)
        z = logits - m
        lse = jnp.log(jnp.sum(jnp.exp(z), axis=1, keepdims=True))
        o_ref[...] = z - lse


def _pool(h_fin, batch3, wl, bl):
    return pl.pallas_call(
        _pool_body,
        grid=(NBLK_P,),
        in_specs=[
            pl.BlockSpec((BN_P, 3), lambda i: (i, 0)),
            pl.BlockSpec((1, 1, BN_P), lambda i: (i, 0, 0)),
            pl.BlockSpec((CC, 3), lambda i: (0, 0)),
            pl.BlockSpec((1, CC), lambda i: (0, 0)),
        ],
        out_specs=pl.BlockSpec((GG, CC), lambda i: (0, 0)),
        out_shape=jax.ShapeDtypeStruct((GG, CC), F32),
        scratch_shapes=[pltpu.VMEM((GG, 4), F32)],
    )(h_fin, batch3, wl, bl)


def _r1(b):
    return b.reshape(1, -1)


def kernel(x, params, edge_index, batch):
    src = edge_index[0]
    dst = edge_index[1]
    pos = x

    zeros3 = jnp.zeros((NN, 3), F32)
    zeros2 = jnp.zeros((NN, 2), F32)
    tdst = jnp.concatenate([zeros3, pos, zeros2], axis=1)

    p1 = params["conv1"]
    h_all = _node_h(x, p1["h"][0]["W"], _r1(p1["h"][0]["b"]),
                    p1["h"][1]["W"], _r1(p1["h"][1]["b"]))

    h_fin = None
    for ci in range(3):
        p = params["conv%d" % (ci + 1)]
        tsrc = jnp.concatenate([h_all, pos, zeros2], axis=1)
        srows, drows = _gather_call(tsrc, tdst, src, dst)
        w1 = p["f"][0]["W"]
        w1p = jnp.concatenate([w1, jnp.zeros((64, 2), F32)], axis=1)
        w1bd = jnp.kron(jnp.eye(16, dtype=F32), w1p.T)
        b1t = jnp.tile(_r1(p["f"][0]["b"]), (1, 16))
        w2t = p["f"][1]["W"].T
        w2pl = jnp.einsum("jk,hc->jhck", jnp.eye(16, dtype=F32),
                          w2t).reshape(1024, 48)
        w2pl = jnp.concatenate([w2pl, jnp.zeros((1024, 80), F32)], axis=1)
        b2pl = jnp.concatenate(
            [jnp.repeat(p["f"][1]["b"], 16), jnp.zeros((80,), F32)])[None, :]
        msg2 = _edge_mlp(srows.reshape(ER, 128), drows.reshape(ER, 128),
                         w1bd, b1t, w2pl, b2pl)
        partial = _scatter_call(dst, msg2)
        if ci < 2:
            pn = params["conv%d" % (ci + 2)]
            h_all = _merge_mlp(
                partial,
                p["g"][0]["W"], _r1(p["g"][0]["b"]),
                p["g"][1]["W"], _r1(p["g"][1]["b"]),
                pn["h"][0]["W"], _r1(pn["h"][0]["b"]),
                pn["h"][1]["W"], _r1(pn["h"][1]["b"]),
                do_relu=True, do_h=True)
        else:
            zb64 = jnp.zeros((1, 64), F32)
            zb3 = jnp.zeros((1, 3), F32)
            zw1 = jnp.zeros((64, 3), F32)
            zw2 = jnp.zeros((3, 64), F32)
            h_fin = _merge_mlp(
                partial,
                p["g"][0]["W"], _r1(p["g"][0]["b"]),
                p["g"][1]["W"], _r1(p["g"][1]["b"]),
                zw1, zb64, zw2, zb3,
                do_relu=False, do_h=False)

    batch3 = batch.reshape(NBLK_P, 1, BN_P)
    return _pool(h_fin, batch3, params["linear"]["W"],
                 _r1(params["linear"]["b"]))


# speedup vs baseline: 1.0519x; 1.0519x over previous
import functools

import jax
import jax.numpy as jnp
from jax import lax
from jax.experimental import pallas as pl
from jax.experimental.pallas import tpu as pltpu
from jax.experimental.pallas import tpu_sc as plsc

NN = 50000
EE = 1600000
GG = 64
CC = 40

NTILES = 32
EPT = EE // NTILES
GCB = 1000
NCHUNK = 10
ECH = EE // NCHUNK
SCB = 1600

_SC_MESH = plsc.VectorSubcoreMesh(
    core_axis_name="c", subcore_axis_name="s", num_cores=2, num_subcores=16)
_SC_PARAMS = pltpu.CompilerParams(use_tc_tiling_on_sc=False,
                                  needs_layout_passes=False)

F32 = jnp.float32


def _gather_body(tsrc, tdst, src_i, dst_i, out_s, out_d,
                 sidx, didx, srows, drows, tdst_sh, sem1, sem2):
    wid = lax.axis_index("s") * 2 + lax.axis_index("c")
    base = wid * EPT

    @pl.when(lax.axis_index("s") == 0)
    def _():
        pltpu.sync_copy(tdst, tdst_sh)

    plsc.subcore_barrier()

    nch = EPT // GCB

    def _cp(k, b):
        off = base + k * GCB
        pltpu.sync_copy(src_i.at[pl.ds(off, GCB)], sidx.at[b])
        pltpu.sync_copy(dst_i.at[pl.ds(off, GCB)], didx.at[b])
        return (pltpu.make_async_copy(tsrc.at[sidx.at[b]], srows.at[b], sem1),
                pltpu.make_async_copy(tdst_sh.at[didx.at[b]], drows.at[b],
                                      sem2))

    def _start(k, b):
        c1, c2 = _cp(k, b)
        c1.start()
        c2.start()

    _start(0, 0)

    def step(k2, carry):
        for b in (0, 1):
            k = 2 * k2 + b
            c1 = pltpu.make_async_copy(tsrc.at[sidx.at[b]], srows.at[b], sem1)
            c2 = pltpu.make_async_copy(tdst_sh.at[didx.at[b]], drows.at[b],
                                       sem2)
            c1.wait()
            c2.wait()

            @pl.when(k + 1 < nch)
            def _():
                _start(k + 1, 1 - b)

            off = base + k * GCB
            pltpu.sync_copy(srows.at[b], out_s.at[pl.ds(off, GCB)])
            pltpu.sync_copy(drows.at[b], out_d.at[pl.ds(off, GCB)])
        return carry

    lax.fori_loop(0, nch // 2, step, 0)


_gather_call = pl.kernel(
    _gather_body,
    out_type=(
        jax.ShapeDtypeStruct((EE, 8), F32),
        jax.ShapeDtypeStruct((EE, 8), F32),
    ),
    mesh=_SC_MESH,
    compiler_params=_SC_PARAMS,
    scratch_types=[
        pltpu.VMEM((2, GCB), jnp.int32),
        pltpu.VMEM((2, GCB), jnp.int32),
        pltpu.VMEM((2, GCB, 8), F32),
        pltpu.VMEM((2, GCB, 8), F32),
        pltpu.VMEM_SHARED((NN, 8), F32),
        pltpu.SemaphoreType.DMA,
        pltpu.SemaphoreType.DMA,
    ],
)


def _scatter_body(dst_i, msg2, out, didx, mvals, agg):
    wid = lax.axis_index("s") * 2 + lax.axis_index("c")

    neg = jnp.full((16,), -jnp.inf, F32)

    def init_step(i, carry):
        agg[pl.ds(i * 16, 16)] = neg
        return carry

    lax.fori_loop(0, NN // 16, init_step, 0)

    @pl.when(wid < 3 * NCHUNK)
    def _():
        ch = wid // NCHUNK
        chunk = wid % NCHUNK

        def outer(k, carry):
            off = chunk * ECH + k * SCB
            pltpu.sync_copy(dst_i.at[pl.ds(off, SCB)], didx)
            pltpu.sync_copy(msg2.at[pl.ds(off // 16, SCB // 16), :], mvals)

            def inner(g, c2):
                d16 = didx[pl.ds(g * 16, 16)]
                m16 = mvals[g, pl.ds(ch * 16, 16)]
                cnt, _ = plsc.scan_count(d16)
                rmin = lax.reduce_min(cnt, (0,))
                rmax = lax.reduce_max(cnt, (0,))

                def rbody(r, c3):
                    sel = cnt == r
                    cur = plsc.load_gather(agg, [d16])
                    newv = jnp.maximum(cur, m16)
                    plsc.store_scatter(agg, [d16], newv, mask=sel)
                    return c3

                lax.fori_loop(rmin, rmax + 1, rbody, 0)
                return c2

            lax.fori_loop(0, SCB // 16, inner, 0)
            return carry

        lax.fori_loop(0, ECH // SCB, outer, 0)

    pltpu.sync_copy(agg, out.at[wid])


_scatter_call = pl.kernel(
    _scatter_body,
    out_type=jax.ShapeDtypeStruct((NTILES, NN), F32),
    mesh=_SC_MESH,
    compiler_params=_SC_PARAMS,
    scratch_types=[
        pltpu.VMEM((SCB,), jnp.int32),
        pltpu.VMEM((SCB // 16, 128), F32),
        pltpu.VMEM((NN,), F32),
    ],
)


def _dot_nt(a, b):
    return lax.dot_general(a, b, (((1,), (1,)), ((), ())),
                           preferred_element_type=F32)


def _node_h_body(x_ref, w1, b1, w2, b2, o_ref):
    xb = x_ref[...]
    hid = jnp.maximum(_dot_nt(xb, w1[...]) + b1[...], 0.0)
    o_ref[...] = _dot_nt(hid, w2[...]) + b2[...]


BN_A = 8192


def _node_h(x, w1, b1, w2, b2):
    grid = (NN + BN_A - 1) // BN_A
    return pl.pallas_call(
        _node_h_body,
        grid=(grid,),
        in_specs=[
            pl.BlockSpec((BN_A, 3), lambda i: (i, 0)),
            pl.BlockSpec((64, 3), lambda i: (0, 0)),
            pl.BlockSpec((1, 64), lambda i: (0, 0)),
            pl.BlockSpec((3, 64), lambda i: (0, 0)),
            pl.BlockSpec((1, 3), lambda i: (0, 0)),
        ],
        out_specs=pl.BlockSpec((BN_A, 3), lambda i: (i, 0)),
        out_shape=jax.ShapeDtypeStruct((NN, 3), F32),
    )(x, w1, b1, w2, b2)


BR = 256
ER = EE // 16


def _edge_mlp_body(s_ref, d_ref, w1, b1, w2, b2, o_ref):
    feat = s_ref[...] - d_ref[...]
    hid = lax.dot_general(feat, w1[...], (((1,), (0,)), ((), ())),
                          preferred_element_type=F32)
    hid = jnp.maximum(hid + b1[...], 0.0)
    msg = lax.dot_general(hid, w2[...], (((1,), (0,)), ((), ())),
                          preferred_element_type=F32)
    o_ref[...] = msg + b2[...]


def _edge_mlp(srows2, drows2, w1bd, b1t, w2pl, b2pl):
    grid = (ER + BR - 1) // BR
    return pl.pallas_call(
        _edge_mlp_body,
        grid=(grid,),
        in_specs=[
            pl.BlockSpec((BR, 128), lambda i: (i, 0)),
            pl.BlockSpec((BR, 128), lambda i: (i, 0)),
            pl.BlockSpec((128, 1024), lambda i: (0, 0)),
            pl.BlockSpec((1, 1024), lambda i: (0, 0)),
            pl.BlockSpec((1024, 128), lambda i: (0, 0)),
            pl.BlockSpec((1, 128), lambda i: (0, 0)),
        ],
        out_specs=pl.BlockSpec((BR, 128), lambda i: (i, 0)),
        out_shape=jax.ShapeDtypeStruct((ER, 128), F32),
    )(srows2, drows2, w1bd, b1t, w2pl, b2pl)


def _merge_mlp_body(p_ref, wg1, bg1, wg2, bg2, wh1, bh1, wh2, bh2, o_ref,
                    *, do_relu, do_h):
    p = p_ref[...]
    a0 = jnp.max(p[0:10], axis=0)
    a1 = jnp.max(p[10:20], axis=0)
    a2 = jnp.max(p[20:30], axis=0)
    agg_t = jnp.concatenate(
        [a0.reshape(1, -1), a1.reshape(1, -1), a2.reshape(1, -1)], axis=0)
    agg_t = jnp.where(jnp.isfinite(agg_t), agg_t, 0.0)
    eye3 = jnp.eye(3, dtype=F32)
    agg = lax.dot_general(agg_t, eye3, (((0,), (0,)), ((), ())),
                          preferred_element_type=F32)
    hid = jnp.maximum(_dot_nt(agg, wg1[...]) + bg1[...], 0.0)
    g = _dot_nt(hid, wg2[...]) + bg2[...]
    if do_relu:
        g = jnp.maximum(g, 0.0)
    if do_h:
        hid2 = jnp.maximum(_dot_nt(g, wh1[...]) + bh1[...], 0.0)
        g = _dot_nt(hid2, wh2[...]) + bh2[...]
    o_ref[...] = g


BN_B = 2048


def _merge_mlp(partial, wg1, bg1, wg2, bg2, wh1, bh1, wh2, bh2,
               do_relu, do_h):
    grid = (NN + BN_B - 1) // BN_B
    body = functools.partial(_merge_mlp_body, do_relu=do_relu, do_h=do_h)
    return pl.pallas_call(
        body,
        grid=(grid,),
        in_specs=[
            pl.BlockSpec((NTILES, BN_B), lambda i: (0, i)),
            pl.BlockSpec((64, 3), lambda i: (0, 0)),
            pl.BlockSpec((1, 64), lambda i: (0, 0)),
            pl.BlockSpec((3, 64), lambda i: (0, 0)),
            pl.BlockSpec((1, 3), lambda i: (0, 0)),
            pl.BlockSpec((64, 3), lambda i: (0, 0)),
            pl.BlockSpec((1, 64), lambda i: (0, 0)),
            pl.BlockSpec((3, 64), lambda i: (0, 0)),
            pl.BlockSpec((1, 3), lambda i: (0, 0)),
        ],
        out_specs=pl.BlockSpec((BN_B, 3), lambda i: (i, 0)),
        out_shape=jax.ShapeDtypeStruct((NN, 3), F32),
    )(partial, wg1, bg1, wg2, bg2, wh1, bh1, wh2, bh2)


BN_P = 2000
NBLK_P = NN // BN_P


def _pool_body(h_ref, b_ref, wl, bl, o_ref, acc):
    i = pl.program_id(0)

    @pl.when(i == 0)
    def _():
        acc[...] = jnp.zeros_like(acc)

    h = h_ref[...]
    bt = b_ref[0, 0, :]
    iota = lax.broadcasted_iota(jnp.int32, (GG, BN_P), 0)
    oneh = (iota == bt[None, :]).astype(F32)
    sums = lax.dot_general(oneh, h, (((1,), (0,)), ((), ())),
                           preferred_element_type=F32)
    cnts = jnp.sum(oneh, axis=1).reshape(GG, 1)
    acc[:, 0:3] += sums
    acc[:, 3:4] += cnts

    @pl.when(i == NBLK_P - 1)
    def _():
        pooled = acc[:, 0:3] / jnp.maximum(acc[:, 3:4], 1.0)
        logits = _dot_nt(pooled, wl[...]) + bl[...]
        m = jnp.max(logits, axis=1, keepdims=True)
        z = logits - m
        lse = jnp.log(jnp.sum(jnp.exp(z), axis=1, keepdims=True))
        o_ref[...] = z - lse


def _pool(h_fin, batch3, wl, bl):
    return pl.pallas_call(
        _pool_body,
        grid=(NBLK_P,),
        in_specs=[
            pl.BlockSpec((BN_P, 3), lambda i: (i, 0)),
            pl.BlockSpec((1, 1, BN_P), lambda i: (i, 0, 0)),
            pl.BlockSpec((CC, 3), lambda i: (0, 0)),
            pl.BlockSpec((1, CC), lambda i: (0, 0)),
        ],
        out_specs=pl.BlockSpec((GG, CC), lambda i: (0, 0)),
        out_shape=jax.ShapeDtypeStruct((GG, CC), F32),
        scratch_shapes=[pltpu.VMEM((GG, 4), F32)],
    )(h_fin, batch3, wl, bl)


def _r1(b):
    return b.reshape(1, -1)


def kernel(x, params, edge_index, batch):
    src = edge_index[0]
    dst = edge_index[1]
    pos = x

    zeros3 = jnp.zeros((NN, 3), F32)
    zeros2 = jnp.zeros((NN, 2), F32)
    tdst = jnp.concatenate([zeros3, pos, zeros2], axis=1)

    p1 = params["conv1"]
    h_all = _node_h(x, p1["h"][0]["W"], _r1(p1["h"][0]["b"]),
                    p1["h"][1]["W"], _r1(p1["h"][1]["b"]))

    h_fin = None
    for ci in range(3):
        p = params["conv%d" % (ci + 1)]
        tsrc = jnp.concatenate([h_all, pos, zeros2], axis=1)
        srows, drows = _gather_call(tsrc, tdst, src, dst)
        w1 = p["f"][0]["W"]
        w1p = jnp.concatenate([w1, jnp.zeros((64, 2), F32)], axis=1)
        w1bd = jnp.kron(jnp.eye(16, dtype=F32), w1p.T)
        b1t = jnp.tile(_r1(p["f"][0]["b"]), (1, 16))
        w2t = p["f"][1]["W"].T
        w2pl = jnp.einsum("jk,hc->jhck", jnp.eye(16, dtype=F32),
                          w2t).reshape(1024, 48)
        w2pl = jnp.concatenate([w2pl, jnp.zeros((1024, 80), F32)], axis=1)
        b2pl = jnp.concatenate(
            [jnp.repeat(p["f"][1]["b"], 16), jnp.zeros((80,), F32)])[None, :]
        msg2 = _edge_mlp(srows.reshape(ER, 128), drows.reshape(ER, 128),
                         w1bd, b1t, w2pl, b2pl)
        partial = _scatter_call(dst, msg2)
        if ci < 2:
            pn = params["conv%d" % (ci + 2)]
            h_all = _merge_mlp(
                partial,
                p["g"][0]["W"], _r1(p["g"][0]["b"]),
                p["g"][1]["W"], _r1(p["g"][1]["b"]),
                pn["h"][0]["W"], _r1(pn["h"][0]["b"]),
                pn["h"][1]["W"], _r1(pn["h"][1]["b"]),
                do_relu=True, do_h=True)
        else:
            zb64 = jnp.zeros((1, 64), F32)
            zb3 = jnp.zeros((1, 3), F32)
            zw1 = jnp.zeros((64, 3), F32)
            zw2 = jnp.zeros((3, 64), F32)
            h_fin = _merge_mlp(
                partial,
                p["g"][0]["W"], _r1(p["g"][0]["b"]),
                p["g"][1]["W"], _r1(p["g"][1]["b"]),
                zw1, zb64, zw2, zb3,
                do_relu=False, do_h=False)

    batch3 = batch.reshape(NBLK_P, 1, BN_P)
    return _pool(h_fin, batch3, params["linear"]["W"],
                 _r1(params["linear"]["b"]))
